# baseline (speedup 1.0000x reference)
import functools

import jax
import jax.numpy as jnp
from jax import lax
from jax.experimental import pallas as pl
from jax.experimental.pallas import tpu as pltpu
from jax.experimental.pallas import tpu_sc as plsc

NUM_NODES = 1000000
EMBED_DIM = 32
BATCH = 16384

NC = 2
NS = 16
NW = NC * NS
B_PER_W = BATCH // NW
CHUNK = 128
NCHUNK = B_PER_W // CHUNK

_mesh = plsc.VectorSubcoreMesh(
    core_axis_name="c", subcore_axis_name="s", num_cores=NC, num_subcores=NS
)


@functools.partial(
    pl.kernel,
    mesh=_mesh,
    out_type=jax.ShapeDtypeStruct((BATCH, EMBED_DIM), jnp.float32),
    scratch_types=[
        pltpu.VMEM((B_PER_W,), jnp.int32),
        pltpu.VMEM((B_PER_W, EMBED_DIM), jnp.float32),
        pltpu.SemaphoreType.DMA,
    ],
    compiler_params=pltpu.CompilerParams(use_tc_tiling_on_sc=False),
)
def _gather_kernel(table_hbm, idx_hbm, out_hbm, idx_v, rows_v, sem):
    wid = lax.axis_index("s") * NC + lax.axis_index("c")
    base = wid * B_PER_W
    pltpu.sync_copy(idx_hbm.at[pl.ds(base, B_PER_W)], idx_v)
    copies = [
        pltpu.async_copy(
            table_hbm.at[idx_v.at[pl.ds(j * CHUNK, CHUNK)]],
            rows_v.at[pl.ds(j * CHUNK, CHUNK)],
            sem,
        )
        for j in range(NCHUNK)
    ]
    for c in copies:
        c.wait()
    pltpu.sync_copy(rows_v, out_hbm.at[pl.ds(base, B_PER_W)])


def kernel(table, batch):
    return _gather_kernel(table, batch.astype(jnp.int32))


# speedup vs baseline: 4.1067x; 4.1067x over previous
import functools

import jax
import jax.numpy as jnp
from jax import lax
from jax.experimental import pallas as pl
from jax.experimental.pallas import tpu as pltpu
from jax.experimental.pallas import tpu_sc as plsc

NUM_NODES = 1000000
EMBED_DIM = 32
BATCH = 16384

NC = 2
NS = 16
NW = NC * NS
B_PER_W = BATCH // NW
DEPTH = 16
GROUPS = B_PER_W // DEPTH

_mesh = plsc.VectorSubcoreMesh(
    core_axis_name="c", subcore_axis_name="s", num_cores=NC, num_subcores=NS
)


@functools.partial(
    pl.kernel,
    mesh=_mesh,
    out_type=jax.ShapeDtypeStruct((EMBED_DIM, BATCH), jnp.float32),
    scratch_types=[
        pltpu.VMEM((B_PER_W,), jnp.int32),
        pltpu.VMEM((DEPTH, EMBED_DIM, 128), jnp.float32),
        pltpu.VMEM((EMBED_DIM, B_PER_W), jnp.float32),
        pltpu.SemaphoreType.DMA((DEPTH,)),
    ],
    compiler_params=pltpu.CompilerParams(needs_layout_passes=False),
)
def _gather_kernel(tbl_hbm, idx_hbm, out_hbm, idx_v, buf_v, out_v, sems):
    wid = lax.axis_index("s") * NC + lax.axis_index("c")
    base = wid * B_PER_W
    pltpu.sync_copy(idx_hbm.at[pl.ds(base, B_PER_W)], idx_v)

    rows_lo = lax.iota(jnp.int32, 16)
    rows_hi = rows_lo + 16

    def col_slice(i):
        return pl.ds(pl.multiple_of((i >> 7) * 128, 128), 128)

    def fire(i, r):
        pltpu.async_copy(
            tbl_hbm.at[:, col_slice(i)], buf_v.at[r], sems.at[r]
        )

    first = idx_v[pl.ds(0, DEPTH)]
    for r in range(DEPTH):
        fire(first[r], r)

    def group(g, _):
        cur = idx_v[pl.ds(g * DEPTH, DEPTH)]
        nxt_start = jnp.minimum((g + 1) * DEPTH, B_PER_W - DEPTH)
        nxt = idx_v[pl.ds(nxt_start, DEPTH)]
        not_last = g < GROUPS - 1
        for r in range(DEPTH):
            j = g * DEPTH + r
            i = cur[r]
            lane_v = jnp.full((16,), i & 127, jnp.int32)
            pltpu.make_async_copy(
                tbl_hbm.at[:, col_slice(i)], buf_v.at[r], sems.at[r]
            ).wait()
            v_lo = plsc.load_gather(buf_v.at[r], [rows_lo, lane_v])
            v_hi = plsc.load_gather(buf_v.at[r], [rows_hi, lane_v])

            @pl.when(not_last)
            def _():
                fire(nxt[r], r)

            col_v = jnp.full((16,), j, jnp.int32)
            plsc.store_scatter(out_v, [rows_lo, col_v], v_lo)
            plsc.store_scatter(out_v, [rows_hi, col_v], v_hi)
        return ()

    lax.fori_loop(0, GROUPS, group, (), unroll=False)
    pltpu.sync_copy(out_v, out_hbm.at[:, pl.ds(base, B_PER_W)])


def kernel(table, batch):
    out_t = _gather_kernel(table.T, batch)
    return out_t.T
